# baseline (speedup 1.0000x reference)
import functools

import jax
import jax.numpy as jnp
from jax import lax
from jax.experimental import pallas as pl
from jax.experimental.pallas import tpu as pltpu
from jax.experimental.pallas import tpu_sc as plsc

NUM_SEG = 512
PAD = NUM_SEG + 16
NC, NS = 2, 16
NW = NC * NS


def _matvec_tc(h, w8, bs):
    n, d = h.shape

    def body(h_ref, w_ref, o_ref):
        o_ref[...] = lax.dot_general(
            w_ref[...], h_ref[...],
            dimension_numbers=(((1,), (1,)), ((), ())),
            preferred_element_type=jnp.float32,
            precision=lax.Precision.DEFAULT)

    return pl.pallas_call(
        body,
        grid=(n // bs,),
        in_specs=[
            pl.BlockSpec((bs, d), lambda i: (i, 0)),
            pl.BlockSpec((8, d), lambda i: (0, 0)),
        ],
        out_specs=pl.BlockSpec((8, bs), lambda i: (0, i)),
        out_shape=jax.ShapeDtypeStruct((8, n), jnp.float32),
    )(h, w8)


def _make_segsum_sc(n):
    chunk = n // NW
    groups = chunk // 16
    mesh = plsc.VectorSubcoreMesh(
        core_axis_name="c", subcore_axis_name="s",
        num_cores=NC, num_subcores=NS)

    @functools.partial(
        pl.kernel,
        out_type=jax.ShapeDtypeStruct((NC, NUM_SEG), jnp.float32),
        mesh=mesh,
        compiler_params=pltpu.CompilerParams(needs_layout_passes=False),
        scratch_types=[
            pltpu.VMEM((chunk,), jnp.float32),
            pltpu.VMEM((chunk + 16,), jnp.int32),
            pltpu.VMEM((PAD,), jnp.float32),
            pltpu.VMEM_SHARED((NS, NUM_SEG), jnp.float32),
            pltpu.VMEM((NS, NUM_SEG), jnp.float32),
        ],
    )
    def segsum(s_hbm, ids_hbm, out_hbm, sv, idv, acc, shared, mbuf):
        cid = lax.axis_index("c")
        sid = lax.axis_index("s")
        wid = sid * NC + cid
        base = wid * chunk

        pltpu.sync_copy(s_hbm.at[pl.ds(base, chunk)], sv)
        pltpu.sync_copy(ids_hbm.at[pl.ds(base, chunk)], idv.at[pl.ds(0, chunk)])
        idv[pl.ds(chunk, 16)] = jnp.full((16,), NUM_SEG, jnp.int32)

        for k in range(PAD // 16):
            acc[pl.ds(k * 16, 16)] = jnp.zeros((16,), jnp.float32)

        lane = lax.iota(jnp.int32, 16)
        rot1 = (lane + 1) & 15
        last = jnp.full((16,), 15, jnp.int32)
        zero = jnp.zeros((16,), jnp.int32)

        def body(g, carry):
            off = g * 16
            vals = sv[pl.ds(off, 16)]
            ids = idv[pl.ds(off, 16)]
            ids_nblk = idv[pl.ds(off + 16, 16)]
            ids_next = jnp.where(
                lane == 15,
                ids_nblk.at[zero].get(mode="promise_in_bounds"),
                ids.at[rot1].get(mode="promise_in_bounds"))
            pref = plsc.cumsum(vals) + carry
            bnd = ids != ids_next
            plsc.addupdate_scatter(acc, [ids], pref, mask=bnd)
            plsc.addupdate_scatter(acc, [ids_next], -pref, mask=bnd)
            return pref.at[last].get(mode="promise_in_bounds")

        lax.fori_loop(0, groups, body, jnp.zeros((16,), jnp.float32))

        pltpu.sync_copy(acc.at[pl.ds(0, NUM_SEG)], shared.at[sid])
        plsc.subcore_barrier()

        @pl.when(sid == 0)
        def _():
            pltpu.sync_copy(shared, mbuf)
            for k in range(NUM_SEG // 16):
                tot = jnp.zeros((16,), jnp.float32)
                for r in range(NS):
                    tot = tot + mbuf[r, pl.ds(k * 16, 16)]
                acc[pl.ds(k * 16, 16)] = tot
            pltpu.sync_copy(acc.at[pl.ds(0, NUM_SEG)], out_hbm.at[cid])

    return segsum


def kernel(h, h_batch, W, b):
    n, d = h.shape
    w8 = jnp.broadcast_to(W, (8, d))
    s8 = _matvec_tc(h, w8, bs=6400)
    s_flat = s8.reshape(8 * n)
    ids = h_batch.astype(jnp.int32)
    partials = _make_segsum_sc(n)(s_flat, ids)
    return partials[0] + partials[1] + b[0]


# speedup vs baseline: 1.1496x; 1.1496x over previous
import functools

import jax
import jax.numpy as jnp
from jax import lax
from jax.experimental import pallas as pl
from jax.experimental.pallas import tpu as pltpu
from jax.experimental.pallas import tpu_sc as plsc

NUM_SEG = 512
PAD = NUM_SEG + 16
NC, NS = 2, 16
NW = NC * NS


def _matvec_tc(h, w8, bs):
    n, d = h.shape

    def body(h_ref, w_ref, o_ref):
        o_ref[...] = lax.dot_general(
            w_ref[...], h_ref[...],
            dimension_numbers=(((1,), (1,)), ((), ())),
            preferred_element_type=jnp.float32,
            precision=lax.Precision.DEFAULT)

    return pl.pallas_call(
        body,
        grid=(n // bs,),
        compiler_params=pltpu.CompilerParams(vmem_limit_bytes=100 * 1024 * 1024),
        in_specs=[
            pl.BlockSpec((bs, d), lambda i: (i, 0)),
            pl.BlockSpec((8, d), lambda i: (0, 0)),
        ],
        out_specs=pl.BlockSpec((8, bs), lambda i: (0, i)),
        out_shape=jax.ShapeDtypeStruct((8, n), jnp.float32),
    )(h, w8)


def _make_segsum_sc(n):
    chunk = n // NW
    groups = chunk // 16
    mesh = plsc.VectorSubcoreMesh(
        core_axis_name="c", subcore_axis_name="s",
        num_cores=NC, num_subcores=NS)

    @functools.partial(
        pl.kernel,
        out_type=jax.ShapeDtypeStruct((NC, NUM_SEG), jnp.float32),
        mesh=mesh,
        compiler_params=pltpu.CompilerParams(needs_layout_passes=False),
        scratch_types=[
            pltpu.VMEM((chunk,), jnp.float32),
            pltpu.VMEM((chunk + 16,), jnp.int32),
            pltpu.VMEM((PAD,), jnp.float32),
            pltpu.VMEM_SHARED((NS, NUM_SEG), jnp.float32),
            pltpu.VMEM((NS, NUM_SEG), jnp.float32),
        ],
    )
    def segsum(s_hbm, ids_hbm, out_hbm, sv, idv, acc, shared, mbuf):
        cid = lax.axis_index("c")
        sid = lax.axis_index("s")
        wid = sid * NC + cid
        base = wid * chunk

        pltpu.sync_copy(s_hbm.at[pl.ds(base, chunk)], sv)
        pltpu.sync_copy(ids_hbm.at[pl.ds(base, chunk)], idv.at[pl.ds(0, chunk)])
        idv[pl.ds(chunk, 16)] = jnp.full((16,), NUM_SEG, jnp.int32)

        for k in range(PAD // 16):
            acc[pl.ds(k * 16, 16)] = jnp.zeros((16,), jnp.float32)

        lane = lax.iota(jnp.int32, 16)
        rot1 = (lane + 1) & 15
        last = jnp.full((16,), 15, jnp.int32)
        zero = jnp.zeros((16,), jnp.int32)

        def group(off, carry):
            vals = sv[pl.ds(off, 16)]
            ids = idv[pl.ds(off, 16)]
            ids_nblk = idv[pl.ds(off + 16, 16)]
            ids_next = jnp.where(
                lane == 15,
                ids_nblk.at[zero].get(mode="promise_in_bounds"),
                ids.at[rot1].get(mode="promise_in_bounds"))
            pref = plsc.cumsum(vals) + carry
            bnd = ids != ids_next
            plsc.addupdate_scatter(acc, [ids], pref, mask=bnd)
            plsc.addupdate_scatter(acc, [ids_next], -pref, mask=bnd)
            return pref.at[last].get(mode="promise_in_bounds")

        quads = groups // 4

        def quad(q, carry):
            off = q * 64
            ids0 = idv[pl.ds(off, 16)]
            ids4 = idv[pl.ds(off + 64, 16)]

            def fast(c):
                tot = (sv[pl.ds(off, 16)] + sv[pl.ds(off + 16, 16)]
                       + sv[pl.ds(off + 32, 16)] + sv[pl.ds(off + 48, 16)])
                return c + jnp.sum(tot)

            def slow(c):
                for u in range(4):
                    c = group(off + u * 16, c)
                return c

            return lax.cond(jnp.any(ids0 != ids4), slow, fast, carry)

        carry = lax.fori_loop(0, quads, quad, jnp.zeros((16,), jnp.float32))
        for g in range(quads * 4, groups):
            carry = group(g * 16, carry)

        pltpu.sync_copy(acc.at[pl.ds(0, NUM_SEG)], shared.at[sid])
        plsc.subcore_barrier()

        @pl.when(sid == 0)
        def _():
            pltpu.sync_copy(shared, mbuf)
            for k in range(NUM_SEG // 16):
                tot = jnp.zeros((16,), jnp.float32)
                for r in range(NS):
                    tot = tot + mbuf[r, pl.ds(k * 16, 16)]
                acc[pl.ds(k * 16, 16)] = tot
            pltpu.sync_copy(acc.at[pl.ds(0, NUM_SEG)], out_hbm.at[cid])

    return segsum


def kernel(h, h_batch, W, b):
    n, d = h.shape
    w8 = jnp.broadcast_to(W, (8, d))
    s8 = _matvec_tc(h, w8, bs=32000)
    s_flat = s8.reshape(8 * n)
    ids = h_batch.astype(jnp.int32)
    partials = _make_segsum_sc(n)(s_flat, ids)
    return partials[0] + partials[1] + b[0]


# speedup vs baseline: 1.1746x; 1.0218x over previous
import functools

import jax
import jax.numpy as jnp
from jax import lax
from jax.experimental import pallas as pl
from jax.experimental.pallas import tpu as pltpu
from jax.experimental.pallas import tpu_sc as plsc

NUM_SEG = 512
PAD = NUM_SEG + 16
NC, NS = 2, 16
NW = NC * NS


def _matvec_tc(h, w8, bs):
    n, d = h.shape

    def body(h_ref, w_ref, o_ref):
        o_ref[...] = lax.dot_general(
            w_ref[...], h_ref[...],
            dimension_numbers=(((1,), (1,)), ((), ())),
            preferred_element_type=jnp.float32,
            precision=lax.Precision.DEFAULT)

    return pl.pallas_call(
        body,
        grid=(n // bs,),
        compiler_params=pltpu.CompilerParams(vmem_limit_bytes=100 * 1024 * 1024),
        in_specs=[
            pl.BlockSpec((bs, d), lambda i: (i, 0)),
            pl.BlockSpec((8, d), lambda i: (0, 0)),
        ],
        out_specs=pl.BlockSpec((8, bs), lambda i: (0, i)),
        out_shape=jax.ShapeDtypeStruct((8, n), jnp.float32),
    )(h, w8)


def _make_segsum_sc(n):
    chunk = n // NW
    groups = chunk // 16
    mesh = plsc.VectorSubcoreMesh(
        core_axis_name="c", subcore_axis_name="s",
        num_cores=NC, num_subcores=NS)

    @functools.partial(
        pl.kernel,
        out_type=jax.ShapeDtypeStruct((NC, NUM_SEG), jnp.float32),
        mesh=mesh,
        compiler_params=pltpu.CompilerParams(needs_layout_passes=False),
        scratch_types=[
            pltpu.VMEM((chunk,), jnp.float32),
            pltpu.VMEM((chunk + 16,), jnp.int32),
            pltpu.VMEM((PAD,), jnp.float32),
            pltpu.VMEM_SHARED((NUM_SEG,), jnp.float32),
            pltpu.VMEM((NUM_SEG,), jnp.int32),
        ],
    )
    def segsum(s_hbm, ids_hbm, out_hbm, sv, idv, acc, shared, iota_v):
        cid = lax.axis_index("c")
        sid = lax.axis_index("s")
        wid = sid * NC + cid
        base = wid * chunk

        pltpu.sync_copy(s_hbm.at[pl.ds(base, chunk)], sv)
        pltpu.sync_copy(ids_hbm.at[pl.ds(base, chunk)], idv.at[pl.ds(0, chunk)])
        idv[pl.ds(chunk, 16)] = jnp.full((16,), NUM_SEG, jnp.int32)

        lane = lax.iota(jnp.int32, 16)
        for k in range(PAD // 16):
            acc[pl.ds(k * 16, 16)] = jnp.zeros((16,), jnp.float32)
        for k in range(NUM_SEG // 16):
            iota_v[pl.ds(k * 16, 16)] = lane + (k * 16)

        @pl.when(sid == 0)
        def _():
            pltpu.sync_copy(acc.at[pl.ds(0, NUM_SEG)], shared)
        plsc.subcore_barrier()

        rot1 = (lane + 1) & 15
        last = jnp.full((16,), 15, jnp.int32)
        zero = jnp.zeros((16,), jnp.int32)

        def group(off, carry):
            vals = sv[pl.ds(off, 16)]
            ids = idv[pl.ds(off, 16)]
            ids_nblk = idv[pl.ds(off + 16, 16)]
            ids_next = jnp.where(
                lane == 15,
                ids_nblk.at[zero].get(mode="promise_in_bounds"),
                ids.at[rot1].get(mode="promise_in_bounds"))
            pref = plsc.cumsum(vals) + carry
            bnd = ids != ids_next
            plsc.addupdate_scatter(acc, [ids], pref, mask=bnd)
            plsc.addupdate_scatter(acc, [ids_next], -pref, mask=bnd)
            return pref.at[last].get(mode="promise_in_bounds")

        quads = groups // 4

        def quad(q, carry):
            off = q * 64
            ids0 = idv[pl.ds(off, 16)]
            ids4 = idv[pl.ds(off + 64, 16)]

            def fast(c):
                tot = (sv[pl.ds(off, 16)] + sv[pl.ds(off + 16, 16)]
                       + sv[pl.ds(off + 32, 16)] + sv[pl.ds(off + 48, 16)])
                return c + jnp.sum(tot)

            def slow(c):
                for u in range(4):
                    c = group(off + u * 16, c)
                return c

            return lax.cond(jnp.any(ids0 != ids4), slow, fast, carry)

        carry = lax.fori_loop(0, quads, quad, jnp.zeros((16,), jnp.float32))
        for g in range(quads * 4, groups):
            carry = group(g * 16, carry)

        pltpu.sync_copy(acc.at[pl.ds(0, NUM_SEG)], shared.at[iota_v], add=True)
        plsc.subcore_barrier()

        @pl.when(sid == 0)
        def _():
            pltpu.sync_copy(shared, acc.at[pl.ds(0, NUM_SEG)])
            pltpu.sync_copy(acc.at[pl.ds(0, NUM_SEG)], out_hbm.at[cid])

    return segsum


def kernel(h, h_batch, W, b):
    n, d = h.shape
    w8 = jnp.broadcast_to(W, (8, d))
    s8 = _matvec_tc(h, w8, bs=32000)
    s_flat = s8.reshape(8 * n)
    ids = h_batch.astype(jnp.int32)
    partials = _make_segsum_sc(n)(s_flat, ids)
    return partials[0] + partials[1] + b[0]
